# baseline (speedup 1.0000x reference)
import functools

import jax
import jax.numpy as jnp
from jax import lax
from jax.experimental import pallas as pl
from jax.experimental.pallas import tpu as pltpu
from jax.experimental.pallas import tpu_sc as plsc

N = 10000
D = 128
BATCH = 100
E = 320000

NC = 2
NS = 16
NW = NC * NS
C = 80
PER_TILE = 10240
NCHUNK = PER_TILE // C
EPAD = PER_TILE * NW
REAL_PER_TILE = E // NW
PADT = PER_TILE - REAL_PER_TILE
NACC = 10112
ZROWS = NACC // NS
FROWS = ZROWS

_mesh = plsc.VectorSubcoreMesh(
    core_axis_name="c", subcore_axis_name="s", num_cores=NC, num_subcores=NS)


@functools.partial(
    pl.kernel,
    out_type=jax.ShapeDtypeStruct((NC, NACC, D), jnp.float32),
    mesh=_mesh,
    scratch_types=[
        pltpu.VMEM((NCHUNK // 2, C), jnp.int32),
        pltpu.VMEM((NCHUNK // 2, C), jnp.int32),
        pltpu.VMEM((3, C, D), jnp.float32),
        pltpu.VMEM_SHARED((NACC, D), jnp.float32),
        pltpu.SemaphoreType.DMA((3,)),
        pltpu.SemaphoreType.DMA((3,)),
    ],
)
def _sc_aggregate(h_hbm, src_hbm, dst_hbm, z_hbm, out_hbm,
                  sidx, didx, rows, acc, gsem, ssem):
    c = lax.axis_index("c")
    s = lax.axis_index("s")
    nh = NCHUNK // 2
    cb = (c * NS + s) * NCHUNK

    zdesc = pltpu.async_copy(z_hbm.at[pl.ds(s * ZROWS, ZROWS)],
                             acc.at[pl.ds(s * ZROWS, ZROWS)], ssem.at[0])

    for half in range(2):
        pltpu.sync_copy(src_hbm.at[pl.ds(cb + half * nh, nh)], sidx)
        pltpu.sync_copy(dst_hbm.at[pl.ds(cb + half * nh, nh)], didx)

        pltpu.async_copy(h_hbm.at[sidx.at[0]], rows.at[0], gsem.at[0])

        if half == 0:
            zdesc.wait()
            plsc.subcore_barrier()

        def body(j, carry):
            b = lax.rem(j, 3)
            nb = lax.rem(j + 1, 3)
            pb = lax.rem(j + 2, 3)

            @pl.when(j > 0)
            def _():
                pltpu.make_async_copy(rows.at[pb], acc.at[didx.at[j - 1]],
                                      ssem.at[pb]).wait()

            @pl.when(j + 1 < nh)
            def _():
                pltpu.async_copy(h_hbm.at[sidx.at[j + 1]], rows.at[nb],
                                 gsem.at[nb])

            pltpu.make_async_copy(h_hbm.at[sidx.at[j]], rows.at[b],
                                  gsem.at[b]).wait()
            pltpu.async_copy(rows.at[b], acc.at[didx.at[j]], ssem.at[b],
                             add=True)
            return carry

        lax.fori_loop(0, nh, body, 0)
        pltpu.make_async_copy(rows.at[lax.rem(nh - 1, 3)],
                              acc.at[didx.at[nh - 1]],
                              ssem.at[lax.rem(nh - 1, 3)]).wait()
    plsc.subcore_barrier()

    pltpu.sync_copy(acc.at[pl.ds(s * FROWS, FROWS)],
                    out_hbm.at[c, pl.ds(s * FROWS, FROWS)])


_ROWS_BLK = 1000


def _embed_body(x_ref, w_ref, o_ref):
    o_ref[...] = jnp.dot(x_ref[...], w_ref[...],
                         preferred_element_type=jnp.float32)


def _embed(x, w):
    return pl.pallas_call(
        _embed_body,
        grid=(N // _ROWS_BLK,),
        in_specs=[
            pl.BlockSpec((_ROWS_BLK, D), lambda i: (i, 0)),
            pl.BlockSpec((D, D), lambda i: (0, 0)),
        ],
        out_specs=pl.BlockSpec((_ROWS_BLK, D), lambda i: (i, 0)),
        out_shape=jax.ShapeDtypeStruct((N, D), jnp.float32),
    )(x, w)


def _res_body(h_ref, wr_ref, bg_ref, br_ref, o_ref):
    o_ref[...] = (jnp.dot(h_ref[...], wr_ref[...],
                          preferred_element_type=jnp.float32)
                  + bg_ref[...] + br_ref[...])


def _res(h, wr, bg, br):
    return pl.pallas_call(
        _res_body,
        grid=(N // _ROWS_BLK,),
        in_specs=[
            pl.BlockSpec((_ROWS_BLK, D), lambda i: (i, 0)),
            pl.BlockSpec((D, D), lambda i: (0, 0)),
            pl.BlockSpec((1, D), lambda i: (0, 0)),
            pl.BlockSpec((1, D), lambda i: (0, 0)),
        ],
        out_specs=pl.BlockSpec((_ROWS_BLK, D), lambda i: (i, 0)),
        out_shape=jax.ShapeDtypeStruct((N, D), jnp.float32),
    )(h, wr, bg.reshape(1, D), br.reshape(1, D))


def _combine_body(s_ref, res_ref, wg_ref, o_ref):
    agg = s_ref[0] + s_ref[1]
    o_ref[...] = (jnp.dot(agg, wg_ref[...], preferred_element_type=jnp.float32)
                  + res_ref[...])


def _combine(s, res, wg):
    return pl.pallas_call(
        _combine_body,
        grid=(N // _ROWS_BLK,),
        in_specs=[
            pl.BlockSpec((NC, _ROWS_BLK, D), lambda i: (0, i, 0)),
            pl.BlockSpec((_ROWS_BLK, D), lambda i: (i, 0)),
            pl.BlockSpec((D, D), lambda i: (0, 0)),
        ],
        out_specs=pl.BlockSpec((_ROWS_BLK, D), lambda i: (i, 0)),
        out_shape=jax.ShapeDtypeStruct((N, D), jnp.float32),
    )(s, res, wg)


def kernel(x, edge_index, batch_size, W_init,
           Wg0, bg0, Wr0, br0,
           Wg1, bg1, Wr1, br1,
           Wg2, bg2, Wr2, br2):
    pad_src = (jnp.arange(NW * PADT, dtype=jnp.int32) % N).reshape(NW, PADT)
    pad_dst = (N + jnp.arange(NW * PADT, dtype=jnp.int32)
               % (NACC - N)).reshape(NW, PADT)
    src = jnp.concatenate(
        [edge_index[0].reshape(NW, REAL_PER_TILE), pad_src], axis=1)
    dst = jnp.concatenate(
        [edge_index[1].reshape(NW, REAL_PER_TILE), pad_dst], axis=1)
    src = src.reshape(EPAD // C, C)
    dst = dst.reshape(EPAD // C, C)
    z = jnp.zeros((NACC, D), jnp.float32)

    h = _embed(x, W_init)
    for (wg, bg, wr, br) in ((Wg0, bg0, Wr0, br0),
                             (Wg1, bg1, Wr1, br1),
                             (Wg2, bg2, Wr2, br2)):
        s = _sc_aggregate(h, src, dst, z)
        res = _res(h, wr, bg, br)
        h = _combine(s, res, wg)
    return h.reshape(BATCH, N // BATCH, D)


# speedup vs baseline: 1.0942x; 1.0942x over previous
import functools

import jax
import jax.numpy as jnp
from jax import lax
from jax.experimental import pallas as pl
from jax.experimental.pallas import tpu as pltpu
from jax.experimental.pallas import tpu_sc as plsc

N = 10000
D = 128
BATCH = 100
E = 320000

NC = 2
NS = 16
NW = NC * NS
C = 128
PER_TILE = 10240
NCHUNK = PER_TILE // C
EPAD = PER_TILE * NW
REAL_PER_TILE = E // NW
PADT = PER_TILE - REAL_PER_TILE
NACC = 10112
ZROWS = NACC // NS
FROWS = ZROWS

_mesh = plsc.VectorSubcoreMesh(
    core_axis_name="c", subcore_axis_name="s", num_cores=NC, num_subcores=NS)


@functools.partial(
    pl.kernel,
    out_type=jax.ShapeDtypeStruct((NC, NACC, D), jnp.float32),
    mesh=_mesh,
    scratch_types=[
        pltpu.VMEM((NCHUNK // 2, C), jnp.int32),
        pltpu.VMEM((NCHUNK // 2, C), jnp.int32),
        pltpu.VMEM((2, C, D), jnp.float32),
        pltpu.VMEM_SHARED((NACC, D), jnp.float32),
        pltpu.SemaphoreType.DMA((2,)),
        pltpu.SemaphoreType.DMA((2,)),
    ],
)
def _sc_aggregate(h_hbm, src_hbm, dst_hbm, z_hbm, out_hbm,
                  sidx, didx, rows, acc, gsem, ssem):
    c = lax.axis_index("c")
    s = lax.axis_index("s")
    nh = NCHUNK // 2
    cb = (c * NS + s) * NCHUNK

    zdesc = pltpu.async_copy(z_hbm.at[pl.ds(s * ZROWS, ZROWS)],
                             acc.at[pl.ds(s * ZROWS, ZROWS)], ssem.at[0])

    for half in range(2):
        pltpu.sync_copy(src_hbm.at[pl.ds(cb + half * nh, nh)], sidx)
        pltpu.sync_copy(dst_hbm.at[pl.ds(cb + half * nh, nh)], didx)

        pltpu.async_copy(h_hbm.at[sidx.at[0]], rows.at[0], gsem.at[0])

        if half == 0:
            zdesc.wait()
            plsc.subcore_barrier()

        def body(j, carry):
            b = lax.rem(j, 2)
            nb = 1 - b

            @pl.when(j > 0)
            def _():
                pltpu.make_async_copy(rows.at[nb], acc.at[didx.at[j - 1]],
                                      ssem.at[nb]).wait()

            @pl.when(j + 1 < nh)
            def _():
                pltpu.async_copy(h_hbm.at[sidx.at[j + 1]], rows.at[nb],
                                 gsem.at[nb])

            pltpu.make_async_copy(h_hbm.at[sidx.at[j]], rows.at[b],
                                  gsem.at[b]).wait()
            pltpu.async_copy(rows.at[b], acc.at[didx.at[j]], ssem.at[b],
                             add=True)
            return carry

        lax.fori_loop(0, nh, body, 0)
        pltpu.make_async_copy(rows.at[lax.rem(nh - 1, 2)],
                              acc.at[didx.at[nh - 1]],
                              ssem.at[lax.rem(nh - 1, 2)]).wait()
    plsc.subcore_barrier()

    pltpu.sync_copy(acc.at[pl.ds(s * FROWS, FROWS)],
                    out_hbm.at[c, pl.ds(s * FROWS, FROWS)])


_ROWS_BLK = 1000


def _embed_body(x_ref, w_ref, o_ref):
    o_ref[...] = jnp.dot(x_ref[...], w_ref[...],
                         preferred_element_type=jnp.float32)


def _embed(x, w):
    return pl.pallas_call(
        _embed_body,
        grid=(N // _ROWS_BLK,),
        in_specs=[
            pl.BlockSpec((_ROWS_BLK, D), lambda i: (i, 0)),
            pl.BlockSpec((D, D), lambda i: (0, 0)),
        ],
        out_specs=pl.BlockSpec((_ROWS_BLK, D), lambda i: (i, 0)),
        out_shape=jax.ShapeDtypeStruct((N, D), jnp.float32),
    )(x, w)


def _res_body(h_ref, wr_ref, bg_ref, br_ref, o_ref):
    o_ref[...] = (jnp.dot(h_ref[...], wr_ref[...],
                          preferred_element_type=jnp.float32)
                  + bg_ref[...] + br_ref[...])


def _res(h, wr, bg, br):
    return pl.pallas_call(
        _res_body,
        grid=(N // _ROWS_BLK,),
        in_specs=[
            pl.BlockSpec((_ROWS_BLK, D), lambda i: (i, 0)),
            pl.BlockSpec((D, D), lambda i: (0, 0)),
            pl.BlockSpec((1, D), lambda i: (0, 0)),
            pl.BlockSpec((1, D), lambda i: (0, 0)),
        ],
        out_specs=pl.BlockSpec((_ROWS_BLK, D), lambda i: (i, 0)),
        out_shape=jax.ShapeDtypeStruct((N, D), jnp.float32),
    )(h, wr, bg.reshape(1, D), br.reshape(1, D))


def _combine_body(s_ref, res_ref, wg_ref, o_ref):
    agg = s_ref[0] + s_ref[1]
    o_ref[...] = (jnp.dot(agg, wg_ref[...], preferred_element_type=jnp.float32)
                  + res_ref[...])


def _combine(s, res, wg):
    return pl.pallas_call(
        _combine_body,
        grid=(N // _ROWS_BLK,),
        in_specs=[
            pl.BlockSpec((NC, _ROWS_BLK, D), lambda i: (0, i, 0)),
            pl.BlockSpec((_ROWS_BLK, D), lambda i: (i, 0)),
            pl.BlockSpec((D, D), lambda i: (0, 0)),
        ],
        out_specs=pl.BlockSpec((_ROWS_BLK, D), lambda i: (i, 0)),
        out_shape=jax.ShapeDtypeStruct((N, D), jnp.float32),
    )(s, res, wg)


def kernel(x, edge_index, batch_size, W_init,
           Wg0, bg0, Wr0, br0,
           Wg1, bg1, Wr1, br1,
           Wg2, bg2, Wr2, br2):
    pad_src = (jnp.arange(NW * PADT, dtype=jnp.int32) % N).reshape(NW, PADT)
    pad_dst = (N + jnp.arange(NW * PADT, dtype=jnp.int32)
               % (NACC - N)).reshape(NW, PADT)
    src = jnp.concatenate(
        [edge_index[0].reshape(NW, REAL_PER_TILE), pad_src], axis=1)
    dst = jnp.concatenate(
        [edge_index[1].reshape(NW, REAL_PER_TILE), pad_dst], axis=1)
    src = src.reshape(EPAD // C, C)
    dst = dst.reshape(EPAD // C, C)
    z = jnp.zeros((NACC, D), jnp.float32)

    h = _embed(x, W_init)
    for (wg, bg, wr, br) in ((Wg0, bg0, Wr0, br0),
                             (Wg1, bg1, Wr1, br1),
                             (Wg2, bg2, Wr2, br2)):
        s = _sc_aggregate(h, src, dst, z)
        res = _res(h, wr, bg, br)
        h = _combine(s, res, wg)
    return h.reshape(BATCH, N // BATCH, D)


# speedup vs baseline: 1.1190x; 1.0226x over previous
import functools

import jax
import jax.numpy as jnp
from jax import lax
from jax.experimental import pallas as pl
from jax.experimental.pallas import tpu as pltpu
from jax.experimental.pallas import tpu_sc as plsc

N = 10000
D = 128
BATCH = 100
E = 320000

NC = 2
NS = 16
NW = NC * NS
C = 128
PER_TILE = 10240
NCHUNK = PER_TILE // C
EPAD = PER_TILE * NW
REAL_PER_TILE = E // NW
PADT = PER_TILE - REAL_PER_TILE
NACC = 10112
ZROWS = NACC // NS
FROWS = ZROWS

_mesh = plsc.VectorSubcoreMesh(
    core_axis_name="c", subcore_axis_name="s", num_cores=NC, num_subcores=NS)


@functools.partial(
    pl.kernel,
    out_type=jax.ShapeDtypeStruct((NC, NACC, D), jnp.float32),
    mesh=_mesh,
    scratch_types=[
        pltpu.VMEM((NCHUNK // 2, C), jnp.int32),
        pltpu.VMEM((NCHUNK // 2, C), jnp.int32),
        pltpu.VMEM((2, C, D), jnp.float32),
        pltpu.VMEM_SHARED((NACC, D), jnp.float32),
        pltpu.SemaphoreType.DMA((2,)),
        pltpu.SemaphoreType.DMA((2,)),
    ],
)
def _sc_aggregate(h_hbm, src_hbm, dst_hbm, z_hbm, out_hbm,
                  sidx, didx, rows, acc, gsem, ssem):
    c = lax.axis_index("c")
    s = lax.axis_index("s")
    nh = NCHUNK // 2
    cb = (c * NS + s) * NCHUNK

    zdesc = pltpu.async_copy(z_hbm.at[pl.ds(s * ZROWS, ZROWS)],
                             acc.at[pl.ds(s * ZROWS, ZROWS)], ssem.at[0])

    for half in range(2):
        pltpu.sync_copy(src_hbm.at[pl.ds(cb + half * nh, nh)], sidx)
        pltpu.sync_copy(dst_hbm.at[pl.ds(cb + half * nh, nh)], didx)

        pltpu.async_copy(h_hbm.at[sidx.at[0]], rows.at[0], gsem.at[0])

        if half == 0:
            zdesc.wait()
            plsc.subcore_barrier()

        def body(j, carry):
            b = lax.rem(j, 2)
            nb = 1 - b

            @pl.when(j > 0)
            def _():
                pltpu.make_async_copy(rows.at[nb], acc.at[didx.at[j - 1]],
                                      ssem.at[nb]).wait()

            @pl.when(j + 1 < nh)
            def _():
                pltpu.async_copy(h_hbm.at[sidx.at[j + 1]], rows.at[nb],
                                 gsem.at[nb])

            pltpu.make_async_copy(h_hbm.at[sidx.at[j]], rows.at[b],
                                  gsem.at[b]).wait()
            pltpu.async_copy(rows.at[b], acc.at[didx.at[j]], ssem.at[b],
                             add=True)
            return carry

        lax.fori_loop(0, nh, body, 0)
        pltpu.make_async_copy(rows.at[lax.rem(nh - 1, 2)],
                              acc.at[didx.at[nh - 1]],
                              ssem.at[lax.rem(nh - 1, 2)]).wait()
    plsc.subcore_barrier()

    pltpu.sync_copy(acc.at[pl.ds(s * FROWS, FROWS)],
                    out_hbm.at[c, pl.ds(s * FROWS, FROWS)])


_ROWS_BLK = 2000


def _embed_body(x_ref, w_ref, o_ref):
    o_ref[...] = jnp.dot(x_ref[...], w_ref[...],
                         preferred_element_type=jnp.float32)


def _embed(x, w):
    return pl.pallas_call(
        _embed_body,
        grid=(N // _ROWS_BLK,),
        in_specs=[
            pl.BlockSpec((_ROWS_BLK, D), lambda i: (i, 0)),
            pl.BlockSpec((D, D), lambda i: (0, 0)),
        ],
        out_specs=pl.BlockSpec((_ROWS_BLK, D), lambda i: (i, 0)),
        out_shape=jax.ShapeDtypeStruct((N, D), jnp.float32),
    )(x, w)


def _res_body(h_ref, wr_ref, bg_ref, br_ref, o_ref):
    o_ref[...] = (jnp.dot(h_ref[...], wr_ref[...],
                          preferred_element_type=jnp.float32)
                  + bg_ref[...] + br_ref[...])


def _res(h, wr, bg, br):
    return pl.pallas_call(
        _res_body,
        grid=(N // _ROWS_BLK,),
        in_specs=[
            pl.BlockSpec((_ROWS_BLK, D), lambda i: (i, 0)),
            pl.BlockSpec((D, D), lambda i: (0, 0)),
            pl.BlockSpec((1, D), lambda i: (0, 0)),
            pl.BlockSpec((1, D), lambda i: (0, 0)),
        ],
        out_specs=pl.BlockSpec((_ROWS_BLK, D), lambda i: (i, 0)),
        out_shape=jax.ShapeDtypeStruct((N, D), jnp.float32),
    )(h, wr, bg.reshape(1, D), br.reshape(1, D))


def _combine_body(s_ref, res_ref, wg_ref, o_ref):
    agg = s_ref[0] + s_ref[1]
    o_ref[...] = (jnp.dot(agg, wg_ref[...], preferred_element_type=jnp.float32)
                  + res_ref[...])


def _combine(s, res, wg):
    return pl.pallas_call(
        _combine_body,
        grid=(N // _ROWS_BLK,),
        in_specs=[
            pl.BlockSpec((NC, _ROWS_BLK, D), lambda i: (0, i, 0)),
            pl.BlockSpec((_ROWS_BLK, D), lambda i: (i, 0)),
            pl.BlockSpec((D, D), lambda i: (0, 0)),
        ],
        out_specs=pl.BlockSpec((_ROWS_BLK, D), lambda i: (i, 0)),
        out_shape=jax.ShapeDtypeStruct((N, D), jnp.float32),
    )(s, res, wg)


def kernel(x, edge_index, batch_size, W_init,
           Wg0, bg0, Wr0, br0,
           Wg1, bg1, Wr1, br1,
           Wg2, bg2, Wr2, br2):
    pad_src = (jnp.arange(NW * PADT, dtype=jnp.int32) % N).reshape(NW, PADT)
    pad_dst = (N + jnp.arange(NW * PADT, dtype=jnp.int32)
               % (NACC - N)).reshape(NW, PADT)
    src = jnp.concatenate(
        [edge_index[0].reshape(NW, REAL_PER_TILE), pad_src], axis=1)
    dst = jnp.concatenate(
        [edge_index[1].reshape(NW, REAL_PER_TILE), pad_dst], axis=1)
    src = src.reshape(EPAD // C, C)
    dst = dst.reshape(EPAD // C, C)
    z = jnp.zeros((NACC, D), jnp.float32)

    h = _embed(x, W_init)
    for (wg, bg, wr, br) in ((Wg0, bg0, Wr0, br0),
                             (Wg1, bg1, Wr1, br1),
                             (Wg2, bg2, Wr2, br2)):
        s = _sc_aggregate(h, src, dst, z)
        res = _res(h, wr, bg, br)
        h = _combine(s, res, wg)
    return h.reshape(BATCH, N // BATCH, D)


# speedup vs baseline: 1.1392x; 1.0181x over previous
import functools

import jax
import jax.numpy as jnp
from jax import lax
from jax.experimental import pallas as pl
from jax.experimental.pallas import tpu as pltpu
from jax.experimental.pallas import tpu_sc as plsc

N = 10000
D = 128
BATCH = 100
E = 320000

NC = 2
NS = 16
NW = NC * NS
C = 128
PER_TILE = 10240
NCHUNK = PER_TILE // C
EPAD = PER_TILE * NW
REAL_PER_TILE = E // NW
PADT = PER_TILE - REAL_PER_TILE
NACC = 10112
ZROWS = NACC // NS
FROWS = ZROWS

_mesh = plsc.VectorSubcoreMesh(
    core_axis_name="c", subcore_axis_name="s", num_cores=NC, num_subcores=NS)


@functools.partial(
    pl.kernel,
    out_type=jax.ShapeDtypeStruct((NC, NACC, D), jnp.float32),
    mesh=_mesh,
    scratch_types=[
        pltpu.VMEM((NCHUNK // 2, C), jnp.int32),
        pltpu.VMEM((NCHUNK // 2, C), jnp.int32),
        pltpu.VMEM((2, C, D), jnp.float32),
        pltpu.VMEM_SHARED((NACC, D), jnp.float32),
        pltpu.SemaphoreType.DMA((2,)),
        pltpu.SemaphoreType.DMA((2,)),
    ],
)
def _sc_aggregate(h_hbm, src_hbm, dst_hbm, out_hbm,
                  sidx, didx, rows, acc, gsem, ssem):
    c = lax.axis_index("c")
    s = lax.axis_index("s")
    nh = NCHUNK // 2
    cb = (c * NS + s) * NCHUNK

    for half in range(2):
        pltpu.sync_copy(src_hbm.at[pl.ds(cb + half * nh, nh)], sidx)
        pltpu.sync_copy(dst_hbm.at[pl.ds(cb + half * nh, nh)], didx)

        pltpu.async_copy(h_hbm.at[sidx.at[0]], rows.at[0], gsem.at[0])

        if half == 0:
            zv = jnp.zeros((16,), jnp.float32)

            def zrow(r, carry):
                for k in range(D // 16):
                    rows[1, r, pl.ds(k * 16, 16)] = zv
                return carry

            lax.fori_loop(0, C, zrow, 0)
            for k in range(ZROWS // C):
                pltpu.sync_copy(
                    rows.at[1],
                    acc.at[pl.ds(s * ZROWS + k * C, C)])
            rem = ZROWS - (ZROWS // C) * C
            pltpu.sync_copy(
                rows.at[1, pl.ds(0, rem)],
                acc.at[pl.ds(s * ZROWS + (ZROWS // C) * C, rem)])
            plsc.subcore_barrier()

        def body(j, carry):
            b = lax.rem(j, 2)
            nb = 1 - b

            @pl.when(j > 0)
            def _():
                pltpu.make_async_copy(rows.at[nb], acc.at[didx.at[j - 1]],
                                      ssem.at[nb]).wait()

            @pl.when(j + 1 < nh)
            def _():
                pltpu.async_copy(h_hbm.at[sidx.at[j + 1]], rows.at[nb],
                                 gsem.at[nb])

            pltpu.make_async_copy(h_hbm.at[sidx.at[j]], rows.at[b],
                                  gsem.at[b]).wait()
            pltpu.async_copy(rows.at[b], acc.at[didx.at[j]], ssem.at[b],
                             add=True)
            return carry

        lax.fori_loop(0, nh, body, 0)
        pltpu.make_async_copy(rows.at[lax.rem(nh - 1, 2)],
                              acc.at[didx.at[nh - 1]],
                              ssem.at[lax.rem(nh - 1, 2)]).wait()
    plsc.subcore_barrier()

    pltpu.sync_copy(acc.at[pl.ds(s * FROWS, FROWS)],
                    out_hbm.at[c, pl.ds(s * FROWS, FROWS)])


_ROWS_BLK = 2000


def _embed_body(x_ref, w_ref, o_ref):
    o_ref[...] = jnp.dot(x_ref[...], w_ref[...],
                         preferred_element_type=jnp.float32)


def _embed(x, w):
    return pl.pallas_call(
        _embed_body,
        grid=(N // _ROWS_BLK,),
        in_specs=[
            pl.BlockSpec((_ROWS_BLK, D), lambda i: (i, 0)),
            pl.BlockSpec((D, D), lambda i: (0, 0)),
        ],
        out_specs=pl.BlockSpec((_ROWS_BLK, D), lambda i: (i, 0)),
        out_shape=jax.ShapeDtypeStruct((N, D), jnp.float32),
    )(x, w)


def _res_body(h_ref, wr_ref, bg_ref, br_ref, o_ref):
    o_ref[...] = (jnp.dot(h_ref[...], wr_ref[...],
                          preferred_element_type=jnp.float32)
                  + bg_ref[...] + br_ref[...])


def _res(h, wr, bg, br):
    return pl.pallas_call(
        _res_body,
        grid=(N // _ROWS_BLK,),
        in_specs=[
            pl.BlockSpec((_ROWS_BLK, D), lambda i: (i, 0)),
            pl.BlockSpec((D, D), lambda i: (0, 0)),
            pl.BlockSpec((1, D), lambda i: (0, 0)),
            pl.BlockSpec((1, D), lambda i: (0, 0)),
        ],
        out_specs=pl.BlockSpec((_ROWS_BLK, D), lambda i: (i, 0)),
        out_shape=jax.ShapeDtypeStruct((N, D), jnp.float32),
    )(h, wr, bg.reshape(1, D), br.reshape(1, D))


def _combine_body(s_ref, res_ref, wg_ref, o_ref):
    agg = s_ref[0] + s_ref[1]
    o_ref[...] = (jnp.dot(agg, wg_ref[...], preferred_element_type=jnp.float32)
                  + res_ref[...])


def _combine(s, res, wg):
    return pl.pallas_call(
        _combine_body,
        grid=(N // _ROWS_BLK,),
        in_specs=[
            pl.BlockSpec((NC, _ROWS_BLK, D), lambda i: (0, i, 0)),
            pl.BlockSpec((_ROWS_BLK, D), lambda i: (i, 0)),
            pl.BlockSpec((D, D), lambda i: (0, 0)),
        ],
        out_specs=pl.BlockSpec((_ROWS_BLK, D), lambda i: (i, 0)),
        out_shape=jax.ShapeDtypeStruct((N, D), jnp.float32),
    )(s, res, wg)


def kernel(x, edge_index, batch_size, W_init,
           Wg0, bg0, Wr0, br0,
           Wg1, bg1, Wr1, br1,
           Wg2, bg2, Wr2, br2):
    pad_src = (jnp.arange(NW * PADT, dtype=jnp.int32) % N).reshape(NW, PADT)
    pad_dst = (N + jnp.arange(NW * PADT, dtype=jnp.int32)
               % (NACC - N)).reshape(NW, PADT)
    src = jnp.concatenate(
        [edge_index[0].reshape(NW, REAL_PER_TILE), pad_src], axis=1)
    dst = jnp.concatenate(
        [edge_index[1].reshape(NW, REAL_PER_TILE), pad_dst], axis=1)
    src = src.reshape(EPAD // C, C)
    dst = dst.reshape(EPAD // C, C)

    h = _embed(x, W_init)
    for (wg, bg, wr, br) in ((Wg0, bg0, Wr0, br0),
                             (Wg1, bg1, Wr1, br1),
                             (Wg2, bg2, Wr2, br2)):
        s = _sc_aggregate(h, src, dst)
        res = _res(h, wr, bg, br)
        h = _combine(s, res, wg)
    return h.reshape(BATCH, N // BATCH, D)


# speedup vs baseline: 1.1463x; 1.0062x over previous
import functools

import jax
import jax.numpy as jnp
from jax import lax
from jax.experimental import pallas as pl
from jax.experimental.pallas import tpu as pltpu
from jax.experimental.pallas import tpu_sc as plsc

N = 10000
D = 128
BATCH = 100
E = 320000

NC = 2
NS = 16
NW = NC * NS
C = 128
PER_TILE = 10240
NCHUNK = PER_TILE // C
EPAD = PER_TILE * NW
REAL_PER_TILE = E // NW
PADT = PER_TILE - REAL_PER_TILE
NACC = 10112
ZROWS = NACC // NS
FROWS = ZROWS

_mesh = plsc.VectorSubcoreMesh(
    core_axis_name="c", subcore_axis_name="s", num_cores=NC, num_subcores=NS)


@functools.partial(
    pl.kernel,
    out_type=jax.ShapeDtypeStruct((NC, NACC, D), jnp.float32),
    mesh=_mesh,
    scratch_types=[
        pltpu.VMEM((NCHUNK // 2, C), jnp.int32),
        pltpu.VMEM((NCHUNK // 2, C), jnp.int32),
        pltpu.VMEM((2, C, D), jnp.float32),
        pltpu.VMEM_SHARED((NACC, D), jnp.float32),
        pltpu.SemaphoreType.DMA((2,)),
        pltpu.SemaphoreType.DMA((2,)),
    ],
)
def _sc_aggregate(h_hbm, src_hbm, dst_hbm, out_hbm,
                  sidx, didx, rows, acc, gsem, ssem):
    c = lax.axis_index("c")
    s = lax.axis_index("s")
    nh = NCHUNK // 2
    cb = (c * NS + s) * NCHUNK

    for half in range(2):
        pltpu.sync_copy(src_hbm.at[pl.ds(cb + half * nh, nh)], sidx)
        pltpu.sync_copy(dst_hbm.at[pl.ds(cb + half * nh, nh)], didx)

        pltpu.async_copy(h_hbm.at[sidx.at[0]], rows.at[0], gsem.at[0])

        if half == 0:
            zv = jnp.zeros((16,), jnp.float32)

            def zrow(r, carry):
                for k in range(D // 16):
                    rows[1, r, pl.ds(k * 16, 16)] = zv
                return carry

            lax.fori_loop(0, C, zrow, 0)
            for k in range(ZROWS // C):
                pltpu.sync_copy(
                    rows.at[1],
                    acc.at[pl.ds(s * ZROWS + k * C, C)])
            rem = ZROWS - (ZROWS // C) * C
            pltpu.sync_copy(
                rows.at[1, pl.ds(0, rem)],
                acc.at[pl.ds(s * ZROWS + (ZROWS // C) * C, rem)])
            plsc.subcore_barrier()

        def body(j, carry):
            b = lax.rem(j, 2)
            nb = 1 - b

            @pl.when(j > 0)
            def _():
                pltpu.make_async_copy(rows.at[nb], acc.at[didx.at[j - 1]],
                                      ssem.at[nb]).wait()

            @pl.when(j + 1 < nh)
            def _():
                pltpu.async_copy(h_hbm.at[sidx.at[j + 1]], rows.at[nb],
                                 gsem.at[nb])

            pltpu.make_async_copy(h_hbm.at[sidx.at[j]], rows.at[b],
                                  gsem.at[b]).wait()
            pltpu.async_copy(rows.at[b], acc.at[didx.at[j]], ssem.at[b],
                             add=True)
            return carry

        lax.fori_loop(0, nh, body, 0)
        pltpu.make_async_copy(rows.at[lax.rem(nh - 1, 2)],
                              acc.at[didx.at[nh - 1]],
                              ssem.at[lax.rem(nh - 1, 2)]).wait()
    plsc.subcore_barrier()

    pltpu.sync_copy(acc.at[pl.ds(s * FROWS, FROWS)],
                    out_hbm.at[c, pl.ds(s * FROWS, FROWS)])


_ROWS_BLK = 2000


def _embed_body(x_ref, w_ref, o_ref):
    o_ref[...] = jnp.dot(x_ref[...], w_ref[...],
                         preferred_element_type=jnp.float32)


def _embed(x, w):
    return pl.pallas_call(
        _embed_body,
        grid=(N // _ROWS_BLK,),
        in_specs=[
            pl.BlockSpec((_ROWS_BLK, D), lambda i: (i, 0)),
            pl.BlockSpec((D, D), lambda i: (0, 0)),
        ],
        out_specs=pl.BlockSpec((_ROWS_BLK, D), lambda i: (i, 0)),
        out_shape=jax.ShapeDtypeStruct((N, D), jnp.float32),
    )(x, w)


def _res_body(h_ref, wr_ref, bg_ref, br_ref, o_ref):
    o_ref[...] = (jnp.dot(h_ref[...], wr_ref[...],
                          preferred_element_type=jnp.float32)
                  + (bg_ref[...] + br_ref[...])[None, :])


def _res(h, wr, bg, br):
    return pl.pallas_call(
        _res_body,
        grid=(N // _ROWS_BLK,),
        in_specs=[
            pl.BlockSpec((_ROWS_BLK, D), lambda i: (i, 0)),
            pl.BlockSpec((D, D), lambda i: (0, 0)),
            pl.BlockSpec((D,), lambda i: (0,)),
            pl.BlockSpec((D,), lambda i: (0,)),
        ],
        out_specs=pl.BlockSpec((_ROWS_BLK, D), lambda i: (i, 0)),
        out_shape=jax.ShapeDtypeStruct((N, D), jnp.float32),
    )(h, wr, bg, br)


def _combine_body(s_ref, res_ref, wg_ref, o_ref):
    agg = s_ref[0] + s_ref[1]
    o_ref[...] = (jnp.dot(agg, wg_ref[...], preferred_element_type=jnp.float32)
                  + res_ref[...])


def _combine(s, res, wg):
    return pl.pallas_call(
        _combine_body,
        grid=(N // _ROWS_BLK,),
        in_specs=[
            pl.BlockSpec((NC, _ROWS_BLK, D), lambda i: (0, i, 0)),
            pl.BlockSpec((_ROWS_BLK, D), lambda i: (i, 0)),
            pl.BlockSpec((D, D), lambda i: (0, 0)),
        ],
        out_specs=pl.BlockSpec((_ROWS_BLK, D), lambda i: (i, 0)),
        out_shape=jax.ShapeDtypeStruct((N, D), jnp.float32),
    )(s, res, wg)


def kernel(x, edge_index, batch_size, W_init,
           Wg0, bg0, Wr0, br0,
           Wg1, bg1, Wr1, br1,
           Wg2, bg2, Wr2, br2):
    npad = EPAD - E
    pad_src = jnp.arange(npad, dtype=jnp.int32) % N
    pad_dst = N + jnp.arange(npad, dtype=jnp.int32) % (NACC - N)
    src = jnp.concatenate([edge_index[0], pad_src]).reshape(EPAD // C, C)
    dst = jnp.concatenate([edge_index[1], pad_dst]).reshape(EPAD // C, C)

    h = _embed(x, W_init)
    for (wg, bg, wr, br) in ((Wg0, bg0, Wr0, br0),
                             (Wg1, bg1, Wr1, br1),
                             (Wg2, bg2, Wr2, br2)):
        s = _sc_aggregate(h, src, dst)
        res = _res(h, wr, bg, br)
        h = _combine(s, res, wg)
    return h.reshape(BATCH, N // BATCH, D)
